# Initial kernel scaffold; baseline (speedup 1.0000x reference)
import jax
import jax.numpy as jnp
from jax import lax
from jax.experimental import pallas as pl
from jax.experimental.pallas import tpu as pltpu
from jax.experimental.pallas import tpu_sc as plsc

EMBED = 32
B_TOTAL = 16384 * 50
NC, NS = 2, 16
NW = NC * NS
B_PER_W = B_TOTAL // NW
CHUNK = 1024
N_CHUNKS = B_PER_W // CHUNK


def _sc_body(table_hbm, idx_hbm, out_hbm, idx_v, rows_v, sem):
    wid = lax.axis_index("s") * NC + lax.axis_index("c")
    base = wid * B_PER_W

    def chunk_step(i, carry):
        off = pl.multiple_of(base + i * CHUNK, CHUNK)
        pltpu.sync_copy(idx_hbm.at[pl.ds(off, CHUNK)], idx_v)
        pltpu.async_copy(table_hbm.at[idx_v], rows_v, sem).wait()
        pltpu.sync_copy(rows_v, out_hbm.at[pl.ds(off, CHUNK)])
        return carry

    lax.fori_loop(0, N_CHUNKS, chunk_step, 0)


def kernel(content_input, table):
    idx = content_input.reshape(-1).astype(jnp.int32)
    mesh = plsc.VectorSubcoreMesh(core_axis_name="c", subcore_axis_name="s")
    out = pl.kernel(
        _sc_body,
        mesh=mesh,
        out_type=jax.ShapeDtypeStruct((B_TOTAL, EMBED), jnp.float32),
        scratch_types=[
            pltpu.VMEM((CHUNK,), jnp.int32),
            pltpu.VMEM((CHUNK, EMBED), jnp.float32),
            pltpu.SemaphoreType.DMA,
        ],
    )(table, idx)
    return out.reshape(content_input.shape + (EMBED,))


# speedup vs baseline: 1.0941x; 1.0941x over previous
import jax
import jax.numpy as jnp
from jax import lax
from jax.experimental import pallas as pl
from jax.experimental.pallas import tpu as pltpu
from jax.experimental.pallas import tpu_sc as plsc

EMBED = 32
B_TOTAL = 16384 * 50
NC, NS = 2, 16
NW = NC * NS
B_PER_W = B_TOTAL // NW
CHUNK = 1024
N_CHUNKS = B_PER_W // CHUNK


def _sc_body(table_hbm, idx_hbm, out_hbm, idx_v, rows_v, sem):
    wid = lax.axis_index("s") * NC + lax.axis_index("c")
    base = wid * B_PER_W

    def chunk_step(i, carry):
        off = pl.multiple_of(base + i * CHUNK, CHUNK)
        pltpu.sync_copy(idx_hbm.at[pl.ds(off, CHUNK)], idx_v)
        pltpu.async_copy(table_hbm.at[idx_v], rows_v, sem).wait()
        pltpu.sync_copy(rows_v, out_hbm.at[pl.ds(off, CHUNK)])
        return carry

    lax.fori_loop(0, N_CHUNKS, chunk_step, 0)


def kernel(content_input, table):
    idx = content_input.reshape(-1).astype(jnp.int32)
    mesh = plsc.VectorSubcoreMesh(core_axis_name="c", subcore_axis_name="s")
    out = pl.kernel(
        _sc_body,
        mesh=mesh,
        compiler_params=pltpu.CompilerParams(use_tc_tiling_on_sc=False),
        out_type=jax.ShapeDtypeStruct((B_TOTAL, EMBED), jnp.float32),
        scratch_types=[
            pltpu.VMEM((CHUNK,), jnp.int32),
            pltpu.VMEM((CHUNK, EMBED), jnp.float32),
            pltpu.SemaphoreType.DMA,
        ],
    )(table, idx)
    return out.reshape(content_input.shape + (EMBED,))


# speedup vs baseline: 1.1134x; 1.0177x over previous
import jax
import jax.numpy as jnp
from jax import lax
from jax.experimental import pallas as pl
from jax.experimental.pallas import tpu as pltpu
from jax.experimental.pallas import tpu_sc as plsc

EMBED = 32
B_TOTAL = 16384 * 50
NC, NS = 2, 16
NW = NC * NS
B_PER_W = B_TOTAL // NW
CHUNK = 800
N_CHUNKS = B_PER_W // CHUNK
NBUF = 4
N_OUTER = N_CHUNKS // NBUF


def _sc_body(table_hbm, idx_hbm, out_hbm, idx_all,
             r0, r1, r2, r3, g0, g1, g2, g3, s0, s1, s2, s3):
    rows = (r0, r1, r2, r3)
    gsem = (g0, g1, g2, g3)
    ssem = (s0, s1, s2, s3)
    wid = lax.axis_index("s") * NC + lax.axis_index("c")
    base = wid * B_PER_W

    pltpu.sync_copy(idx_hbm.at[pl.ds(wid * N_CHUNKS, N_CHUNKS)], idx_all)

    for b in range(NBUF):
        pltpu.async_copy(table_hbm.at[idx_all.at[b]], rows[b], gsem[b])

    def outer(i2, carry):
        for b in range(NBUF):
            c = i2 * NBUF + b
            off = pl.multiple_of(base + c * CHUNK, CHUNK)
            pltpu.make_async_copy(table_hbm.at[idx_all.at[c]], rows[b],
                                  gsem[b]).wait()
            pltpu.async_copy(rows[b], out_hbm.at[pl.ds(off, CHUNK)], ssem[b])

            @pl.when(i2 < N_OUTER - 1)
            def _():
                pltpu.make_async_copy(rows[b],
                                      out_hbm.at[pl.ds(off, CHUNK)],
                                      ssem[b]).wait()
                pltpu.async_copy(table_hbm.at[idx_all.at[c + NBUF]],
                                 rows[b], gsem[b])
        return carry

    lax.fori_loop(0, N_OUTER, outer, 0)

    for b in range(NBUF):
        c = (N_OUTER - 1) * NBUF + b
        off = pl.multiple_of(base + c * CHUNK, CHUNK)
        pltpu.make_async_copy(rows[b], out_hbm.at[pl.ds(off, CHUNK)],
                              ssem[b]).wait()


def kernel(content_input, table):
    idx = content_input.reshape(NW * N_CHUNKS, CHUNK).astype(jnp.int32)
    mesh = plsc.VectorSubcoreMesh(core_axis_name="c", subcore_axis_name="s")
    out = pl.kernel(
        _sc_body,
        mesh=mesh,
        compiler_params=pltpu.CompilerParams(use_tc_tiling_on_sc=False),
        out_type=jax.ShapeDtypeStruct((B_TOTAL, EMBED), jnp.float32),
        scratch_types=(
            [pltpu.VMEM((N_CHUNKS, CHUNK), jnp.int32)]
            + [pltpu.VMEM((CHUNK, EMBED), jnp.float32) for _ in range(NBUF)]
            + [pltpu.SemaphoreType.DMA for _ in range(2 * NBUF)]
        ),
    )(table, idx)
    return out.reshape(content_input.shape + (EMBED,))


# speedup vs baseline: 1.8111x; 1.6267x over previous
import jax
import jax.numpy as jnp
from jax import lax
from jax.experimental import pallas as pl
from jax.experimental.pallas import tpu as pltpu
from jax.experimental.pallas import tpu_sc as plsc

SEQS, SLEN, EMBED = 16384, 50, 32
NC, NS = 2, 16
NW = NC * NS
S_PER_W = SEQS // NW
CHUNK = 16
N_CHUNKS = S_PER_W // CHUNK
NBUF = 4
N_OUTER = N_CHUNKS // NBUF


def _sc_body(table_hbm, idx_hbm, out_hbm, idx_all,
             r0, r1, r2, r3, g0, g1, g2, g3, s0, s1, s2, s3):
    rows = (r0, r1, r2, r3)
    gsem = (g0, g1, g2, g3)
    ssem = (s0, s1, s2, s3)
    wid = lax.axis_index("s") * NC + lax.axis_index("c")
    base = wid * S_PER_W

    pltpu.sync_copy(idx_hbm.at[pl.ds(wid * N_CHUNKS, N_CHUNKS)], idx_all)

    def gather(c, b):
        pltpu.async_copy(table_hbm.at[idx_all.at[c]], rows[b], gsem[b])

    def gather_wait(c, b):
        pltpu.make_async_copy(table_hbm.at[idx_all.at[c]], rows[b],
                              gsem[b]).wait()

    def store(c, b):
        for s in range(CHUNK):
            pltpu.async_copy(rows[b].at[pl.ds(s * SLEN, SLEN)],
                             out_hbm.at[base + c * CHUNK + s], ssem[b])

    def store_wait(c, b):
        for s in range(CHUNK):
            pltpu.make_async_copy(rows[b].at[pl.ds(s * SLEN, SLEN)],
                                  out_hbm.at[base + c * CHUNK + s],
                                  ssem[b]).wait()

    for b in range(NBUF):
        gather(b, b)

    def outer(i2, carry):
        for b in range(NBUF):
            c = i2 * NBUF + b
            gather_wait(c, b)
            store(c, b)

            @pl.when(i2 < N_OUTER - 1)
            def _():
                store_wait(c, b)
                gather(c + NBUF, b)
        return carry

    lax.fori_loop(0, N_OUTER, outer, 0)

    for b in range(NBUF):
        store_wait((N_OUTER - 1) * NBUF + b, b)


def kernel(content_input, table):
    mesh = plsc.VectorSubcoreMesh(core_axis_name="c", subcore_axis_name="s")
    return pl.kernel(
        _sc_body,
        mesh=mesh,
        compiler_params=pltpu.CompilerParams(use_tc_tiling_on_sc=False),
        out_type=jax.ShapeDtypeStruct((SEQS, SLEN, EMBED), jnp.float32),
        scratch_types=(
            [pltpu.VMEM((N_CHUNKS, CHUNK * SLEN), jnp.int32)]
            + [pltpu.VMEM((CHUNK * SLEN, EMBED), jnp.float32)
               for _ in range(NBUF)]
            + [pltpu.SemaphoreType.DMA for _ in range(2 * NBUF)]
        ),
    )(table, content_input.astype(jnp.int32).reshape(NW * N_CHUNKS,
                                                       CHUNK * SLEN))


# speedup vs baseline: 1.8941x; 1.0458x over previous
import jax
import jax.numpy as jnp
from jax import lax
from jax.experimental import pallas as pl
from jax.experimental.pallas import tpu as pltpu
from jax.experimental.pallas import tpu_sc as plsc

SEQS, SLEN, EMBED = 16384, 50, 32
NC, NS = 2, 16
NW = NC * NS
S_PER_W = SEQS // NW
NBUF = 2
N_OUTER = SLEN // NBUF


def _sc_body(table_hbm, idx_hbm, out_hbm, idx_all, r0, r1, g0, g1, s0, s1):
    rows = (r0, r1)
    gsem = (g0, g1)
    ssem = (s0, s1)
    wid = lax.axis_index("s") * NC + lax.axis_index("c")
    base = wid * S_PER_W

    for p in range(SLEN):
        pltpu.sync_copy(idx_hbm.at[p, pl.ds(base, S_PER_W)], idx_all.at[p])

    def gather(p, b):
        pltpu.async_copy(table_hbm.at[idx_all.at[p]], rows[b], gsem[b])

    def gather_wait(p, b):
        pltpu.make_async_copy(table_hbm.at[idx_all.at[p]], rows[b],
                              gsem[b]).wait()

    def store(p, b):
        pltpu.async_copy(rows[b], out_hbm.at[p, pl.ds(base, S_PER_W)],
                         ssem[b])

    def store_wait(p, b):
        pltpu.make_async_copy(rows[b], out_hbm.at[p, pl.ds(base, S_PER_W)],
                              ssem[b]).wait()

    for b in range(NBUF):
        gather(b, b)

    def outer(i2, carry):
        for b in range(NBUF):
            p = i2 * NBUF + b
            gather_wait(p, b)
            store(p, b)

            @pl.when(i2 < N_OUTER - 1)
            def _():
                store_wait(p, b)
                gather(p + NBUF, b)
        return carry

    lax.fori_loop(0, N_OUTER, outer, 0)

    for b in range(NBUF):
        store_wait((N_OUTER - 1) * NBUF + b, b)


def kernel(content_input, table):
    mesh = plsc.VectorSubcoreMesh(core_axis_name="c", subcore_axis_name="s")
    out = pl.kernel(
        _sc_body,
        mesh=mesh,
        compiler_params=pltpu.CompilerParams(use_tc_tiling_on_sc=False),
        out_type=jax.ShapeDtypeStruct((SLEN, SEQS, EMBED), jnp.float32),
        scratch_types=(
            [pltpu.VMEM((SLEN, S_PER_W), jnp.int32)]
            + [pltpu.VMEM((S_PER_W, EMBED), jnp.float32) for _ in range(NBUF)]
            + [pltpu.SemaphoreType.DMA for _ in range(2 * NBUF)]
        ),
    )(table, content_input.astype(jnp.int32).T)
    return out.transpose(1, 0, 2)


# speedup vs baseline: 1.8989x; 1.0025x over previous
import jax
import jax.numpy as jnp
from jax import lax
from jax.experimental import pallas as pl
from jax.experimental.pallas import tpu as pltpu
from jax.experimental.pallas import tpu_sc as plsc

SEQS, SLEN, EMBED = 16384, 50, 32
NC, NS = 2, 16
NW = NC * NS
S_PER_W = SEQS // NW
NBUF = 5
N_OUTER = SLEN // NBUF


def _sc_body(table_hbm, idx_hbm, out_hbm, idx_all, r0, r1, r2, r3, r4,
             g0, g1, g2, g3, g4, s0, s1, s2, s3, s4):
    rows = (r0, r1, r2, r3, r4)
    gsem = (g0, g1, g2, g3, g4)
    ssem = (s0, s1, s2, s3, s4)
    wid = lax.axis_index("s") * NC + lax.axis_index("c")
    base = wid * S_PER_W

    for p in range(SLEN):
        pltpu.sync_copy(idx_hbm.at[p, pl.ds(base, S_PER_W)], idx_all.at[p])

    def gather(p, b):
        pltpu.async_copy(table_hbm.at[idx_all.at[p]], rows[b], gsem[b])

    def gather_wait(p, b):
        pltpu.make_async_copy(table_hbm.at[idx_all.at[p]], rows[b],
                              gsem[b]).wait()

    def store(p, b):
        pltpu.async_copy(rows[b], out_hbm.at[p, pl.ds(base, S_PER_W)],
                         ssem[b])

    def store_wait(p, b):
        pltpu.make_async_copy(rows[b], out_hbm.at[p, pl.ds(base, S_PER_W)],
                              ssem[b]).wait()

    for b in range(NBUF):
        gather(b, b)

    def outer(i2, carry):
        for b in range(NBUF):
            p = i2 * NBUF + b
            gather_wait(p, b)
            store(p, b)

            @pl.when(i2 < N_OUTER - 1)
            def _():
                store_wait(p, b)
                gather(p + NBUF, b)
        return carry

    lax.fori_loop(0, N_OUTER, outer, 0)

    for b in range(NBUF):
        store_wait((N_OUTER - 1) * NBUF + b, b)


def kernel(content_input, table):
    mesh = plsc.VectorSubcoreMesh(core_axis_name="c", subcore_axis_name="s")
    out = pl.kernel(
        _sc_body,
        mesh=mesh,
        compiler_params=pltpu.CompilerParams(use_tc_tiling_on_sc=False),
        out_type=jax.ShapeDtypeStruct((SLEN, SEQS, EMBED), jnp.float32),
        scratch_types=(
            [pltpu.VMEM((SLEN, S_PER_W), jnp.int32)]
            + [pltpu.VMEM((S_PER_W, EMBED), jnp.float32) for _ in range(NBUF)]
            + [pltpu.SemaphoreType.DMA for _ in range(2 * NBUF)]
        ),
    )(table, content_input.astype(jnp.int32).T)
    return out.transpose(1, 0, 2)


# speedup vs baseline: 1.9443x; 1.0239x over previous
import jax
import jax.numpy as jnp
from jax import lax
from jax.experimental import pallas as pl
from jax.experimental.pallas import tpu as pltpu
from jax.experimental.pallas import tpu_sc as plsc

SEQS, SLEN, EMBED = 16384, 50, 32
NC, NS = 2, 16
NW = NC * NS
S_PER_W = SEQS // NW
NBUF = 5
N_OUTER = SLEN // NBUF


def _sc_body(table_hbm, idx_hbm, out_hbm, idx_all, r0, r1, r2, r3, r4,
             g0, g1, g2, g3, g4, s0, s1, s2, s3, s4):
    rows = (r0, r1, r2, r3, r4)
    gsem = (g0, g1, g2, g3, g4)
    ssem = (s0, s1, s2, s3, s4)
    wid = lax.axis_index("s") * NC + lax.axis_index("c")
    base = wid * S_PER_W

    pltpu.sync_copy(idx_hbm.at[pl.ds(0, SLEN), pl.ds(base, S_PER_W)],
                    idx_all)

    def gather(p, b):
        pltpu.async_copy(table_hbm.at[idx_all.at[p]], rows[b], gsem[b])

    def gather_wait(p, b):
        pltpu.make_async_copy(table_hbm.at[idx_all.at[p]], rows[b],
                              gsem[b]).wait()

    def store(p, b):
        pltpu.async_copy(rows[b], out_hbm.at[p, pl.ds(base, S_PER_W)],
                         ssem[b])

    def store_wait(p, b):
        pltpu.make_async_copy(rows[b], out_hbm.at[p, pl.ds(base, S_PER_W)],
                              ssem[b]).wait()

    for b in range(NBUF):
        gather(b, b)

    def outer(i2, carry):
        for b in range(NBUF):
            p = i2 * NBUF + b
            gather_wait(p, b)
            store(p, b)

            @pl.when(i2 < N_OUTER - 1)
            def _():
                store_wait(p, b)
                gather(p + NBUF, b)
        return carry

    lax.fori_loop(0, N_OUTER, outer, 0)

    for b in range(NBUF):
        store_wait((N_OUTER - 1) * NBUF + b, b)


def kernel(content_input, table):
    mesh = plsc.VectorSubcoreMesh(core_axis_name="c", subcore_axis_name="s")
    out = pl.kernel(
        _sc_body,
        mesh=mesh,
        compiler_params=pltpu.CompilerParams(use_tc_tiling_on_sc=False),
        out_type=jax.ShapeDtypeStruct((SLEN, SEQS, EMBED), jnp.float32),
        scratch_types=(
            [pltpu.VMEM((SLEN, S_PER_W), jnp.int32)]
            + [pltpu.VMEM((S_PER_W, EMBED), jnp.float32) for _ in range(NBUF)]
            + [pltpu.SemaphoreType.DMA for _ in range(2 * NBUF)]
        ),
    )(table, content_input.astype(jnp.int32).T)
    return out.transpose(1, 0, 2)


# speedup vs baseline: 1.9457x; 1.0007x over previous
import jax
import jax.numpy as jnp
from jax import lax
from jax.experimental import pallas as pl
from jax.experimental.pallas import tpu as pltpu
from jax.experimental.pallas import tpu_sc as plsc

SEQS, SLEN, EMBED = 16384, 50, 32
NC, NS = 2, 16
NW = NC * NS
S_PER_W = SEQS // NW
HALF = S_PER_W // 2
NBUF = 10
N_CHUNKS = SLEN * 2
N_OUTER = N_CHUNKS // NBUF


def _sc_body(table_hbm, idx_hbm, out_hbm, idx_all,
             r0, r1, r2, r3, r4, r5, r6, r7, r8, r9,
             g0, g1, g2, g3, g4, g5, g6, g7, g8, g9,
             s0, s1, s2, s3, s4, s5, s6, s7, s8, s9):
    rows = (r0, r1, r2, r3, r4, r5, r6, r7, r8, r9)
    gsem = (g0, g1, g2, g3, g4, g5, g6, g7, g8, g9)
    ssem = (s0, s1, s2, s3, s4, s5, s6, s7, s8, s9)
    wid = lax.axis_index("s") * NC + lax.axis_index("c")
    base = wid * S_PER_W

    pltpu.sync_copy(idx_hbm.at[pl.ds(0, SLEN), pl.ds(base, S_PER_W)],
                    idx_all)

    def _idx(c):
        return idx_all.at[c // 2, pl.ds((c % 2) * HALF, HALF)]

    def _dst(c):
        return out_hbm.at[c // 2, pl.ds(base + (c % 2) * HALF, HALF)]

    def gather(c, b):
        pltpu.async_copy(table_hbm.at[_idx(c)], rows[b], gsem[b])

    def gather_wait(c, b):
        pltpu.make_async_copy(table_hbm.at[_idx(c)], rows[b],
                              gsem[b]).wait()

    def store(c, b):
        pltpu.async_copy(rows[b], _dst(c), ssem[b])

    def store_wait(c, b):
        pltpu.make_async_copy(rows[b], _dst(c), ssem[b]).wait()

    for b in range(NBUF):
        gather(b, b)

    def outer(i2, carry):
        for b in range(NBUF):
            c = i2 * NBUF + b
            gather_wait(c, b)
            store(c, b)

            @pl.when(i2 < N_OUTER - 1)
            def _():
                store_wait(c, b)
                gather(c + NBUF, b)
        return carry

    lax.fori_loop(0, N_OUTER, outer, 0)

    for b in range(NBUF):
        store_wait((N_OUTER - 1) * NBUF + b, b)


def kernel(content_input, table):
    mesh = plsc.VectorSubcoreMesh(core_axis_name="c", subcore_axis_name="s")
    out = pl.kernel(
        _sc_body,
        mesh=mesh,
        compiler_params=pltpu.CompilerParams(use_tc_tiling_on_sc=False),
        out_type=jax.ShapeDtypeStruct((SLEN, SEQS, EMBED), jnp.float32),
        scratch_types=(
            [pltpu.VMEM((SLEN, S_PER_W), jnp.int32)]
            + [pltpu.VMEM((HALF, EMBED), jnp.float32) for _ in range(NBUF)]
            + [pltpu.SemaphoreType.DMA for _ in range(2 * NBUF)]
        ),
    )(table, content_input.astype(jnp.int32).T)
    return out.transpose(1, 0, 2)
